# baseline (speedup 1.0000x reference)
import functools
import math

import jax
import jax.numpy as jnp
from jax import lax
from jax.experimental import pallas as pl
from jax.experimental.pallas import tpu as pltpu
from jax.experimental.pallas import tpu_sc as plsc

D_EMB = 64
SCALE = math.sqrt(D_EMB)

_info = plsc.get_sparse_core_info()
_NC, _NS, _L = _info.num_cores, _info.num_subcores, _info.num_lanes
_NW = _NC * _NS


def _make_gather(B: int, V: int, D: int, C: int):
  assert B % (_NW * 2 * C) == 0 and C % 8 == 0 and D % _L == 0
  b_per_w = B // _NW
  n_chunks = b_per_w // C
  n_outer = n_chunks // 2
  mesh = plsc.VectorSubcoreMesh(core_axis_name="c", subcore_axis_name="s")

  @functools.partial(
      pl.kernel,
      mesh=mesh,
      out_type=jax.ShapeDtypeStruct((B, D), jnp.float32),
      compiler_params=pltpu.CompilerParams(use_tc_tiling_on_sc=False),
      scratch_types=[
          [pltpu.VMEM((C,), jnp.int32)] * 2,
          [pltpu.VMEM((C, D), jnp.float32)] * 2,
          [pltpu.VMEM((C, D), jnp.float32)] * 2,
          [pltpu.SemaphoreType.DMA] * 2,
          [pltpu.SemaphoreType.DMA] * 2,
      ],
  )
  def gather_kernel(table_hbm, idx_hbm, out_hbm, idx_v, rows_in, rows_out,
                    gsem, ssem):
    wid = lax.axis_index("s") * _NC + lax.axis_index("c")
    base = wid * b_per_w

    for b in (0, 1):
      pltpu.sync_copy(idx_hbm.at[pl.ds(base + b * C, C)], idx_v[b])
      pltpu.async_copy(table_hbm.at[idx_v[b]], rows_in[b], gsem[b])

    def outer_body(go, carry):
      for b in (0, 1):
        g = 2 * go + b
        off = base + g * C
        pltpu.make_async_copy(table_hbm.at[idx_v[b]], rows_in[b],
                              gsem[b]).wait()
        @pl.when(go > 0)
        def _():
          pltpu.make_async_copy(rows_out[b], out_hbm.at[pl.ds(base, C)],
                                ssem[b]).wait()

        def scale_row(j, c2):
          for k in range(D // _L):
            sl = pl.ds(k * _L, _L)
            rows_out[b][j, sl] = rows_in[b][j, sl] * SCALE
          return c2

        lax.fori_loop(0, C, scale_row, 0, unroll=4)

        pltpu.async_copy(rows_out[b], out_hbm.at[pl.ds(off, C)], ssem[b])

        @pl.when(go < n_outer - 1)
        def _():
          noff = off + 2 * C
          pltpu.sync_copy(idx_hbm.at[pl.ds(noff, C)], idx_v[b])
          pltpu.async_copy(table_hbm.at[idx_v[b]], rows_in[b], gsem[b])

      return carry

    lax.fori_loop(0, n_outer, outer_body, 0)

    for b in (0, 1):
      pltpu.make_async_copy(rows_out[b], out_hbm.at[pl.ds(base, C)],
                            ssem[b]).wait()

  return gather_kernel


def kernel(seq, table):
  bsz, hist = seq.shape
  B = bsz * hist
  V, D = table.shape
  idx = seq.reshape(B)
  out = _make_gather(B, V, D, C=400)(table, idx)
  return out.reshape(bsz, hist, D)
